# Initial kernel scaffold; baseline (speedup 1.0000x reference)
import jax
import jax.numpy as jnp
from jax.experimental import pallas as pl


def kernel(x, W):
    raise NotImplementedError("write your pallas kernel here")


# speedup vs baseline: 1.1548x; 1.1548x over previous
import functools

import jax
import jax.numpy as jnp
from jax import lax
from jax.experimental import pallas as pl
from jax.experimental.pallas import tpu as pltpu
from jax.experimental.pallas import tpu_sc as plsc

D = 256
NVALS = 4 * 8192
NW = 32
VPW = NVALS // NW
CHUNK = 128
NCHUNK = VPW // CHUNK


@functools.partial(
    pl.kernel,
    out_type=jax.ShapeDtypeStruct((NVALS, 4, D), jnp.float32),
    mesh=plsc.VectorSubcoreMesh(core_axis_name="c", subcore_axis_name="s"),
    scratch_types=[
        pltpu.VMEM((VPW,), jnp.int32),
        pltpu.VMEM((VPW,), jnp.int32),
        pltpu.VMEM((VPW,), jnp.int32),
        pltpu.VMEM((VPW,), jnp.int32),
        pltpu.VMEM((VPW,), jnp.int32),
        pltpu.VMEM((CHUNK, D), jnp.float32),
        pltpu.SemaphoreType.DMA,
    ],
)
def _emb_kernel(xi_hbm, w_hbm, out_hbm, xi_v, i0_v, i1_v, i2_v, i3_v,
                rows_v, sem):
    wid = lax.axis_index("s") * 2 + lax.axis_index("c")
    vbase = wid * VPW

    pltpu.sync_copy(xi_hbm.at[pl.ds(vbase, VPW)], xi_v)

    idx_refs = (i0_v, i1_v, i2_v, i3_v)

    def build_idx(g, carry):
        v = xi_v[pl.ds(g * 16, 16)]
        for j in range(4):
            byte = lax.shift_right_logical(v, jnp.int32(8 * j)) & 0xFF
            idx_refs[j][pl.ds(g * 16, 16)] = byte
        return carry

    lax.fori_loop(0, VPW // 16, build_idx, 0)

    def emit(c, carry):
        base = vbase + c * CHUNK
        for j in range(4):
            pltpu.async_copy(
                w_hbm.at[idx_refs[j].at[pl.ds(c * CHUNK, CHUNK)]],
                rows_v, sem).wait()
            pltpu.sync_copy(rows_v, out_hbm.at[pl.ds(base, CHUNK), j])
        return carry

    lax.fori_loop(0, NCHUNK, emit, 0)


def kernel(x, W):
    xi = lax.bitcast_convert_type(x, jnp.int32).reshape(-1)
    out = _emb_kernel(xi, W)
    return out.reshape(x.shape[0], x.shape[1], 4 * D)
